# Initial kernel scaffold; baseline (speedup 1.0000x reference)
import jax
import jax.numpy as jnp
from jax.experimental import pallas as pl


def kernel(edge_index, user_emb, item_emb):
    raise NotImplementedError("write your pallas kernel here")


# speedup vs baseline: 45.3088x; 45.3088x over previous
import functools

import jax
import jax.numpy as jnp
from jax import lax
from jax.experimental import pallas as pl
from jax.experimental.pallas import tpu as pltpu
from jax.experimental.pallas import tpu_sc as plsc

NU = 50000
NI = 50000
D = 32
NLAYERS = 3
E = 1600000

NSUB = 16
NPAD = 50176
TRASH = 50100
R = NPAD // NSUB

CH = 128
GRP = 8
GRPL = 4
CPT = 784
NGRP = CPT // GRP
NGRPL = CPT // GRPL
NCHUNK = NSUB * CPT
E_PAD = NCHUNK * CH
SR = GRPL * CH

_mesh = plsc.VectorSubcoreMesh(core_axis_name="c", subcore_axis_name="s")


def _f32(*shape):
    return jax.ShapeDtypeStruct(shape, jnp.float32)


@functools.partial(
    pl.kernel,
    out_type=(_f32(NPAD), _f32(NPAD)),
    mesh=_mesh,
    scratch_types=[
        pltpu.VMEM((GRP, CH), jnp.int32),
        pltpu.VMEM((CH,), jnp.float32),
        pltpu.VMEM((R,), jnp.float32),
        pltpu.VMEM_SHARED((NPAD,), jnp.float32),
    ],
)
def _deg_kernel(eu_ref, ei_ref, z1_ref, du_ref, di_ref, idx_v, ones_v, zb_v, acc):
    cid = lax.axis_index("c")
    sid = lax.axis_index("s")
    for k in range(CH // 16):
        ones_v[pl.ds(16 * k, 16)] = jnp.ones((16,), jnp.float32)
    pltpu.sync_copy(z1_ref, zb_v)
    pltpu.sync_copy(zb_v, acc.at[pl.ds(sid * R, R)])
    plsc.subcore_barrier()

    def run(e_ref, out_ref):
        def body(g, carry):
            cb = sid * CPT + g * GRP
            pltpu.sync_copy(e_ref.at[pl.ds(cb, GRP)], idx_v)
            for j in range(GRP):
                pltpu.sync_copy(ones_v, acc.at[idx_v.at[j]], add=True)
            return carry

        lax.fori_loop(0, NGRP, body, 0)
        plsc.subcore_barrier()
        pltpu.sync_copy(acc.at[pl.ds(sid * R, R)], zb_v)
        pltpu.sync_copy(zb_v, out_ref.at[pl.ds(sid * R, R)])

    @pl.when(cid == 0)
    def _():
        run(eu_ref, du_ref)

    @pl.when(cid == 1)
    def _():
        run(ei_ref, di_ref)


@functools.partial(
    pl.kernel,
    out_type=(_f32(NPAD, D), _f32(NPAD, D)),
    mesh=_mesh,
    scratch_types=[
        pltpu.VMEM((GRPL, CH), jnp.int32),
        pltpu.VMEM((GRPL, CH), jnp.int32),
        pltpu.VMEM((SR, D), jnp.float32),
        pltpu.VMEM_SHARED((NPAD, D), jnp.float32),
        pltpu.SemaphoreType.DMA,
    ],
    compiler_params=pltpu.CompilerParams(use_tc_tiling_on_sc=False),
)
def _layer_kernel(yu_ref, yi_ref, eu_ref, ei_ref, z2_ref, su_ref, si_ref,
                  gidx_v, sidx_v, rows_v, acc, sem):
    cid = lax.axis_index("c")
    sid = lax.axis_index("s")
    nfull, tail = R // SR, R % SR
    pltpu.sync_copy(z2_ref, rows_v)
    for k in range(nfull):
        pltpu.sync_copy(rows_v, acc.at[pl.ds(sid * R + k * SR, SR)])
    if tail:
        pltpu.sync_copy(rows_v.at[pl.ds(0, tail)],
                        acc.at[pl.ds(sid * R + nfull * SR, tail)])
    plsc.subcore_barrier()

    def run(tab_ref, ge_ref, se_ref, out_ref):
        def body(g, carry):
            cb = sid * CPT + g * GRPL
            pltpu.sync_copy(ge_ref.at[pl.ds(cb, GRPL)], gidx_v)
            pltpu.sync_copy(se_ref.at[pl.ds(cb, GRPL)], sidx_v)
            cps = [pltpu.async_copy(tab_ref.at[gidx_v.at[j]],
                                    rows_v.at[pl.ds(j * CH, CH)], sem)
                   for j in range(GRPL)]
            for cp in cps:
                cp.wait()
            for j in range(GRPL):
                pltpu.sync_copy(rows_v.at[pl.ds(j * CH, CH)],
                                acc.at[sidx_v.at[j]], add=True)
            return carry

        lax.fori_loop(0, NGRPL, body, 0)
        plsc.subcore_barrier()
        for k in range(nfull):
            base = sid * R + k * SR
            pltpu.sync_copy(acc.at[pl.ds(base, SR)], rows_v)
            pltpu.sync_copy(rows_v, out_ref.at[pl.ds(base, SR)])
        if tail:
            base = sid * R + nfull * SR
            pltpu.sync_copy(acc.at[pl.ds(base, tail)], rows_v.at[pl.ds(0, tail)])
            pltpu.sync_copy(rows_v.at[pl.ds(0, tail)], out_ref.at[pl.ds(base, tail)])

    @pl.when(cid == 0)
    def _():
        run(yi_ref, ei_ref, eu_ref, su_ref)

    @pl.when(cid == 1)
    def _():
        run(yu_ref, eu_ref, ei_ref, si_ref)


def _prep_body(du_ref, di_ref, emu_ref, emi_ref, yu_ref, yi_ref, d2u_ref, d2i_ref):
    for dref, eref, yref, d2ref in ((du_ref, emu_ref, yu_ref, d2u_ref),
                                    (di_ref, emi_ref, yi_ref, d2i_ref)):
        deg = dref[...]
        pos = deg > 0.0
        safe = jnp.maximum(deg, 1.0)
        dinv = jnp.where(pos, lax.rsqrt(safe), 0.0)
        d2ref[...] = jnp.where(pos, 1.0 / safe, 0.0)
        yref[...] = eref[...] * dinv


def _scale_body(su_ref, si_ref, d2u_ref, d2i_ref, zu_ref, zi_ref,
                yu_o, yi_o, zu_o, zi_o):
    for s, d2, z, yo, zo in ((su_ref, d2u_ref, zu_ref, yu_o, zu_o),
                             (si_ref, d2i_ref, zi_ref, yi_o, zi_o)):
        y = s[...] * d2[...]
        yo[...] = y
        zo[...] = z[...] + y


def _final_body(su_ref, si_ref, d2u_ref, d2i_ref, zu_ref, zi_ref,
                du_ref, di_ref, emu_ref, emi_ref, ou_ref, oi_ref):
    for s, d2, z, dg, em, o in (
            (su_ref, d2u_ref, zu_ref, du_ref, emu_ref, ou_ref),
            (si_ref, d2i_ref, zi_ref, di_ref, emi_ref, oi_ref)):
        zf = z[...] + s[...] * d2[...]
        v = jnp.where(dg[...] > 0.0, zf, em[...])
        n2 = jnp.sum(v * v, axis=1, keepdims=True)
        o[...] = v / jnp.maximum(jnp.sqrt(n2), 1e-12)


TCG = 32
BR = NPAD // TCG
_w = pl.BlockSpec((BR, D), lambda i: (i, 0))
_c = pl.BlockSpec((BR, 1), lambda i: (i, 0))

_prep = pl.pallas_call(
    _prep_body,
    grid=(TCG,),
    in_specs=[_c, _c, _w, _w],
    out_specs=(_w, _w, _c, _c),
    out_shape=(_f32(NPAD, D), _f32(NPAD, D), _f32(NPAD, 1), _f32(NPAD, 1)),
)

_scale = pl.pallas_call(
    _scale_body,
    grid=(TCG,),
    in_specs=[_w, _w, _c, _c, _w, _w],
    out_specs=(_w, _w, _w, _w),
    out_shape=(_f32(NPAD, D), _f32(NPAD, D), _f32(NPAD, D), _f32(NPAD, D)),
)

_final = pl.pallas_call(
    _final_body,
    grid=(TCG,),
    in_specs=[_w, _w, _c, _c, _w, _w, _c, _c, _w, _w],
    out_specs=(_w, _w),
    out_shape=(_f32(NPAD, D), _f32(NPAD, D)),
)


def kernel(edge_index, user_emb, item_emb):
    eu = edge_index[0]
    ei = edge_index[1]
    pad = jnp.full((E_PAD - E,), TRASH, dtype=jnp.int32)
    eu2 = jnp.concatenate([eu, pad]).reshape(NCHUNK, CH)
    ei2 = jnp.concatenate([ei, pad]).reshape(NCHUNK, CH)
    padrows = jnp.zeros((NPAD - NU, D), jnp.float32)
    emu = jnp.concatenate([user_emb, padrows], axis=0)
    emi = jnp.concatenate([item_emb, padrows], axis=0)
    z1 = jnp.zeros((R,), jnp.float32)
    z2 = jnp.zeros((SR, D), jnp.float32)

    du, di = _deg_kernel(eu2, ei2, z1)
    du2 = du.reshape(NPAD, 1)
    di2 = di.reshape(NPAD, 1)
    yu, yi, d2u, d2i = _prep(du2, di2, emu, emi)
    zu, zi = yu, yi
    for _ in range(NLAYERS - 1):
        su, si = _layer_kernel(yu, yi, eu2, ei2, z2)
        yu, yi, zu, zi = _scale(su, si, d2u, d2i, zu, zi)
    su, si = _layer_kernel(yu, yi, eu2, ei2, z2)
    ou, oi = _final(su, si, d2u, d2i, zu, zi, du2, di2, emu, emi)
    return (ou[:NU], oi[:NI])


# speedup vs baseline: 49.1888x; 1.0856x over previous
import functools

import jax
import jax.numpy as jnp
from jax import lax
from jax.experimental import pallas as pl
from jax.experimental.pallas import tpu as pltpu
from jax.experimental.pallas import tpu_sc as plsc

NU = 50000
NI = 50000
D = 32
NLAYERS = 3
E = 1600000

NSUB = 16
NPAD = 50176
TRASH = 50100
R = NPAD // NSUB

CH = 128
GRP = 8
GRPL = 2
CPT = 784
NGRP = CPT // GRP
NGRPL = CPT // GRPL
NCHUNK = NSUB * CPT
E_PAD = NCHUNK * CH
SR = GRPL * CH

_mesh = plsc.VectorSubcoreMesh(core_axis_name="c", subcore_axis_name="s")


def _f32(*shape):
    return jax.ShapeDtypeStruct(shape, jnp.float32)


@functools.partial(
    pl.kernel,
    out_type=(_f32(NPAD), _f32(NPAD)),
    mesh=_mesh,
    scratch_types=[
        pltpu.VMEM((GRP, CH), jnp.int32),
        pltpu.VMEM((CH,), jnp.float32),
        pltpu.VMEM((R,), jnp.float32),
        pltpu.VMEM_SHARED((NPAD,), jnp.float32),
    ],
)
def _deg_kernel(eu_ref, ei_ref, z1_ref, du_ref, di_ref, idx_v, ones_v, zb_v, acc):
    cid = lax.axis_index("c")
    sid = lax.axis_index("s")
    for k in range(CH // 16):
        ones_v[pl.ds(16 * k, 16)] = jnp.ones((16,), jnp.float32)
    pltpu.sync_copy(z1_ref, zb_v)
    pltpu.sync_copy(zb_v, acc.at[pl.ds(sid * R, R)])
    plsc.subcore_barrier()

    def run(e_ref, out_ref):
        def body(g, carry):
            cb = sid * CPT + g * GRP
            pltpu.sync_copy(e_ref.at[pl.ds(cb, GRP)], idx_v)
            for j in range(GRP):
                pltpu.sync_copy(ones_v, acc.at[idx_v.at[j]], add=True)
            return carry

        lax.fori_loop(0, NGRP, body, 0)
        plsc.subcore_barrier()
        pltpu.sync_copy(acc.at[pl.ds(sid * R, R)], zb_v)
        pltpu.sync_copy(zb_v, out_ref.at[pl.ds(sid * R, R)])

    @pl.when(cid == 0)
    def _():
        run(eu_ref, du_ref)

    @pl.when(cid == 1)
    def _():
        run(ei_ref, di_ref)


@functools.partial(
    pl.kernel,
    out_type=(_f32(NPAD, D), _f32(NPAD, D)),
    mesh=_mesh,
    scratch_types=[
        [pltpu.VMEM((GRPL, CH), jnp.int32) for _ in range(3)],
        [pltpu.VMEM((GRPL, CH), jnp.int32) for _ in range(3)],
        [pltpu.VMEM((SR, D), jnp.float32) for _ in range(3)],
        pltpu.VMEM_SHARED((NPAD, D), jnp.float32),
        [pltpu.SemaphoreType.DMA for _ in range(3)],
    ],
    compiler_params=pltpu.CompilerParams(use_tc_tiling_on_sc=False),
)
def _layer_kernel(yu_ref, yi_ref, eu_ref, ei_ref, z2_ref, su_ref, si_ref,
                  gidx_v, sidx_v, rows_v, acc, gsem):
    cid = lax.axis_index("c")
    sid = lax.axis_index("s")
    nfull, tail = R // SR, R % SR
    pltpu.sync_copy(z2_ref, rows_v[0])
    for k in range(nfull):
        pltpu.sync_copy(rows_v[0], acc.at[pl.ds(sid * R + k * SR, SR)])
    if tail:
        pltpu.sync_copy(rows_v[0].at[pl.ds(0, tail)],
                        acc.at[pl.ds(sid * R + nfull * SR, tail)])
    plsc.subcore_barrier()

    def run(tab_ref, ge_ref, se_ref, out_ref):
        def load_and_fire(kk, s):
            cb = sid * CPT + kk * GRPL
            pltpu.sync_copy(ge_ref.at[pl.ds(cb, GRPL)], gidx_v[s])
            pltpu.sync_copy(se_ref.at[pl.ds(cb, GRPL)], sidx_v[s])
            for j in range(GRPL):
                pltpu.async_copy(tab_ref.at[gidx_v[s].at[j]],
                                 rows_v[s].at[pl.ds(j * CH, CH)], gsem[s])

        def drain_and_scatter(s):
            for j in range(GRPL):
                pltpu.make_async_copy(tab_ref.at[gidx_v[s].at[j]],
                                      rows_v[s].at[pl.ds(j * CH, CH)],
                                      gsem[s]).wait()
            for j in range(GRPL):
                pltpu.sync_copy(rows_v[s].at[pl.ds(j * CH, CH)],
                                acc.at[sidx_v[s].at[j]], add=True)

        load_and_fire(0, 0)
        load_and_fire(1, 1)

        def body(t, carry):
            for d in range(3):
                k = 3 * t + d
                load_and_fire(k + 2, (d + 2) % 3)
                drain_and_scatter(d)
            return carry

        lax.fori_loop(0, (NGRPL - 2) // 3, body, 0)
        drain_and_scatter((NGRPL - 2) % 3)
        drain_and_scatter((NGRPL - 1) % 3)
        plsc.subcore_barrier()
        for k in range(nfull):
            base = sid * R + k * SR
            pltpu.sync_copy(acc.at[pl.ds(base, SR)], rows_v[k % 3])
            pltpu.sync_copy(rows_v[k % 3], out_ref.at[pl.ds(base, SR)])
        if tail:
            base = sid * R + nfull * SR
            pltpu.sync_copy(acc.at[pl.ds(base, tail)], rows_v[0].at[pl.ds(0, tail)])
            pltpu.sync_copy(rows_v[0].at[pl.ds(0, tail)],
                            out_ref.at[pl.ds(base, tail)])

    @pl.when(cid == 0)
    def _():
        run(yi_ref, ei_ref, eu_ref, su_ref)

    @pl.when(cid == 1)
    def _():
        run(yu_ref, eu_ref, ei_ref, si_ref)


def _prep_body(du_ref, di_ref, emu_ref, emi_ref, yu_ref, yi_ref, d2u_ref, d2i_ref):
    for dref, eref, yref, d2ref in ((du_ref, emu_ref, yu_ref, d2u_ref),
                                    (di_ref, emi_ref, yi_ref, d2i_ref)):
        deg = dref[...]
        pos = deg > 0.0
        safe = jnp.maximum(deg, 1.0)
        dinv = jnp.where(pos, lax.rsqrt(safe), 0.0)
        d2ref[...] = jnp.where(pos, 1.0 / safe, 0.0)
        yref[...] = eref[...] * dinv


def _scale_body(su_ref, si_ref, d2u_ref, d2i_ref, zu_ref, zi_ref,
                yu_o, yi_o, zu_o, zi_o):
    for s, d2, z, yo, zo in ((su_ref, d2u_ref, zu_ref, yu_o, zu_o),
                             (si_ref, d2i_ref, zi_ref, yi_o, zi_o)):
        y = s[...] * d2[...]
        yo[...] = y
        zo[...] = z[...] + y


def _final_body(su_ref, si_ref, d2u_ref, d2i_ref, zu_ref, zi_ref,
                du_ref, di_ref, emu_ref, emi_ref, ou_ref, oi_ref):
    for s, d2, z, dg, em, o in (
            (su_ref, d2u_ref, zu_ref, du_ref, emu_ref, ou_ref),
            (si_ref, d2i_ref, zi_ref, di_ref, emi_ref, oi_ref)):
        zf = z[...] + s[...] * d2[...]
        v = jnp.where(dg[...] > 0.0, zf, em[...])
        n2 = jnp.sum(v * v, axis=1, keepdims=True)
        o[...] = v / jnp.maximum(jnp.sqrt(n2), 1e-12)


TCG = 32
BR = NPAD // TCG
_w = pl.BlockSpec((BR, D), lambda i: (i, 0))
_c = pl.BlockSpec((BR, 1), lambda i: (i, 0))

_prep = pl.pallas_call(
    _prep_body,
    grid=(TCG,),
    in_specs=[_c, _c, _w, _w],
    out_specs=(_w, _w, _c, _c),
    out_shape=(_f32(NPAD, D), _f32(NPAD, D), _f32(NPAD, 1), _f32(NPAD, 1)),
)

_scale = pl.pallas_call(
    _scale_body,
    grid=(TCG,),
    in_specs=[_w, _w, _c, _c, _w, _w],
    out_specs=(_w, _w, _w, _w),
    out_shape=(_f32(NPAD, D), _f32(NPAD, D), _f32(NPAD, D), _f32(NPAD, D)),
)

_final = pl.pallas_call(
    _final_body,
    grid=(TCG,),
    in_specs=[_w, _w, _c, _c, _w, _w, _c, _c, _w, _w],
    out_specs=(_w, _w),
    out_shape=(_f32(NPAD, D), _f32(NPAD, D)),
)


def kernel(edge_index, user_emb, item_emb):
    eu = edge_index[0]
    ei = edge_index[1]
    pad = jnp.full((E_PAD - E,), TRASH, dtype=jnp.int32)
    eu2 = jnp.concatenate([eu, pad]).reshape(NCHUNK, CH)
    ei2 = jnp.concatenate([ei, pad]).reshape(NCHUNK, CH)
    padrows = jnp.zeros((NPAD - NU, D), jnp.float32)
    emu = jnp.concatenate([user_emb, padrows], axis=0)
    emi = jnp.concatenate([item_emb, padrows], axis=0)
    z1 = jnp.zeros((R,), jnp.float32)
    z2 = jnp.zeros((SR, D), jnp.float32)

    du, di = _deg_kernel(eu2, ei2, z1)
    du2 = du.reshape(NPAD, 1)
    di2 = di.reshape(NPAD, 1)
    yu, yi, d2u, d2i = _prep(du2, di2, emu, emi)
    zu, zi = yu, yi
    for _ in range(NLAYERS - 1):
        su, si = _layer_kernel(yu, yi, eu2, ei2, z2)
        yu, yi, zu, zi = _scale(su, si, d2u, d2i, zu, zi)
    su, si = _layer_kernel(yu, yi, eu2, ei2, z2)
    ou, oi = _final(su, si, d2u, d2i, zu, zi, du2, di2, emu, emi)
    return (ou[:NU], oi[:NI])


# speedup vs baseline: 67.8098x; 1.3786x over previous
import functools

import jax
import jax.numpy as jnp
from jax import lax
from jax.experimental import pallas as pl
from jax.experimental.pallas import tpu as pltpu
from jax.experimental.pallas import tpu_sc as plsc

NU = 50000
NI = 50000
D = 32
NLAYERS = 3
E = 1600000

NSUB = 16
NPAD = 50176
TRASH = 50100
R = NPAD // NSUB

CH = 128
GRP = 8
GRPL = 2
CPT = 784
NGRP = CPT // GRP
NGRPL = CPT // GRPL
NCHUNK = NSUB * CPT
E_PAD = NCHUNK * CH
SR = GRPL * CH

_mesh = plsc.VectorSubcoreMesh(core_axis_name="c", subcore_axis_name="s")


def _f32(*shape):
    return jax.ShapeDtypeStruct(shape, jnp.float32)


@functools.partial(
    pl.kernel,
    out_type=(_f32(NPAD), _f32(NPAD)),
    mesh=_mesh,
    scratch_types=[
        pltpu.VMEM((GRP, CH), jnp.int32),
        pltpu.VMEM((CH,), jnp.float32),
        pltpu.VMEM((R,), jnp.float32),
        pltpu.VMEM_SHARED((NPAD,), jnp.float32),
    ],
)
def _deg_kernel(eu_ref, ei_ref, z1_ref, du_ref, di_ref, idx_v, ones_v, zb_v, acc):
    cid = lax.axis_index("c")
    sid = lax.axis_index("s")
    for k in range(CH // 16):
        ones_v[pl.ds(16 * k, 16)] = jnp.ones((16,), jnp.float32)
    pltpu.sync_copy(z1_ref, zb_v)
    pltpu.sync_copy(zb_v, acc.at[pl.ds(sid * R, R)])
    plsc.subcore_barrier()

    def run(e_ref, out_ref):
        def body(g, carry):
            cb = sid * CPT + g * GRP
            pltpu.sync_copy(e_ref.at[pl.ds(cb, GRP)], idx_v)
            for j in range(GRP):
                pltpu.sync_copy(ones_v, acc.at[idx_v.at[j]], add=True)
            return carry

        lax.fori_loop(0, NGRP, body, 0)
        plsc.subcore_barrier()
        pltpu.sync_copy(acc.at[pl.ds(sid * R, R)], zb_v)
        pltpu.sync_copy(zb_v, out_ref.at[pl.ds(sid * R, R)])

    @pl.when(cid == 0)
    def _():
        run(eu_ref, du_ref)

    @pl.when(cid == 1)
    def _():
        run(ei_ref, di_ref)


@functools.partial(
    pl.kernel,
    out_type=(_f32(NPAD, D), _f32(NPAD, D)),
    mesh=_mesh,
    scratch_types=[
        [pltpu.VMEM((GRPL, CH), jnp.int32) for _ in range(3)],
        [pltpu.VMEM((GRPL, CH), jnp.int32) for _ in range(3)],
        [pltpu.VMEM((SR, D), jnp.float32) for _ in range(3)],
        pltpu.VMEM_SHARED((NPAD, D), jnp.float32),
        [pltpu.SemaphoreType.DMA for _ in range(3)],
        [pltpu.SemaphoreType.DMA for _ in range(3)],
    ],
    compiler_params=pltpu.CompilerParams(use_tc_tiling_on_sc=False),
)
def _layer_kernel(yu_ref, yi_ref, eu_ref, ei_ref, z2_ref, su_ref, si_ref,
                  gidx_v, sidx_v, rows_v, acc, gsem, isem):
    cid = lax.axis_index("c")
    sid = lax.axis_index("s")
    nfull, tail = R // SR, R % SR
    pltpu.sync_copy(z2_ref, rows_v[0])
    for k in range(nfull):
        pltpu.sync_copy(rows_v[0], acc.at[pl.ds(sid * R + k * SR, SR)])
    if tail:
        pltpu.sync_copy(rows_v[0].at[pl.ds(0, tail)],
                        acc.at[pl.ds(sid * R + nfull * SR, tail)])
    plsc.subcore_barrier()

    def run(tab_ref, ge_ref, se_ref, out_ref):
        def load_idx(kk, s):
            cb = sid * CPT + kk * GRPL
            pltpu.async_copy(ge_ref.at[pl.ds(cb, GRPL)], gidx_v[s], isem[s])
            pltpu.async_copy(se_ref.at[pl.ds(cb, GRPL)], sidx_v[s], isem[s])

        def wait_idx(s):
            pltpu.make_async_copy(ge_ref.at[pl.ds(0, GRPL)], gidx_v[s], isem[s]).wait()
            pltpu.make_async_copy(ge_ref.at[pl.ds(0, GRPL)], sidx_v[s], isem[s]).wait()

        def fire(s):
            for j in range(GRPL):
                pltpu.async_copy(tab_ref.at[gidx_v[s].at[j]],
                                 rows_v[s].at[pl.ds(j * CH, CH)], gsem[s])

        def drain_and_scatter(s):
            for j in range(GRPL):
                pltpu.make_async_copy(tab_ref.at[gidx_v[s].at[j]],
                                      rows_v[s].at[pl.ds(j * CH, CH)],
                                      gsem[s]).wait()
            for j in range(GRPL):
                pltpu.sync_copy(rows_v[s].at[pl.ds(j * CH, CH)],
                                acc.at[sidx_v[s].at[j]], add=True)

        load_idx(0, 0)
        load_idx(1, 1)
        wait_idx(0)
        fire(0)

        def body(t, carry):
            for d in range(3):
                k = 3 * t + d
                s0, s1, s2 = d, (d + 1) % 3, (d + 2) % 3
                load_idx(k + 2, s2)
                wait_idx(s1)
                fire(s1)
                drain_and_scatter(s0)
            return carry

        lax.fori_loop(0, (NGRPL - 2) // 3, body, 0)
        s0, s1 = (NGRPL - 2) % 3, (NGRPL - 1) % 3
        wait_idx(s1)
        fire(s1)
        drain_and_scatter(s0)
        drain_and_scatter(s1)
        plsc.subcore_barrier()
        for k in range(nfull):
            base = sid * R + k * SR
            pltpu.sync_copy(acc.at[pl.ds(base, SR)], rows_v[k % 3])
            pltpu.sync_copy(rows_v[k % 3], out_ref.at[pl.ds(base, SR)])
        if tail:
            base = sid * R + nfull * SR
            pltpu.sync_copy(acc.at[pl.ds(base, tail)], rows_v[0].at[pl.ds(0, tail)])
            pltpu.sync_copy(rows_v[0].at[pl.ds(0, tail)],
                            out_ref.at[pl.ds(base, tail)])

    @pl.when(cid == 0)
    def _():
        run(yi_ref, ei_ref, eu_ref, su_ref)

    @pl.when(cid == 1)
    def _():
        run(yu_ref, eu_ref, ei_ref, si_ref)


def _prep_body(du_ref, di_ref, emu_ref, emi_ref, yu_ref, yi_ref, d2u_ref, d2i_ref):
    for dref, eref, yref, d2ref in ((du_ref, emu_ref, yu_ref, d2u_ref),
                                    (di_ref, emi_ref, yi_ref, d2i_ref)):
        deg = dref[...]
        pos = deg > 0.0
        safe = jnp.maximum(deg, 1.0)
        dinv = jnp.where(pos, lax.rsqrt(safe), 0.0)
        d2ref[...] = jnp.where(pos, 1.0 / safe, 0.0)
        yref[...] = eref[...] * dinv


def _scale_body(su_ref, si_ref, d2u_ref, d2i_ref, zu_ref, zi_ref,
                yu_o, yi_o, zu_o, zi_o):
    for s, d2, z, yo, zo in ((su_ref, d2u_ref, zu_ref, yu_o, zu_o),
                             (si_ref, d2i_ref, zi_ref, yi_o, zi_o)):
        y = s[...] * d2[...]
        yo[...] = y
        zo[...] = z[...] + y


def _final_body(su_ref, si_ref, d2u_ref, d2i_ref, zu_ref, zi_ref,
                du_ref, di_ref, emu_ref, emi_ref, ou_ref, oi_ref):
    for s, d2, z, dg, em, o in (
            (su_ref, d2u_ref, zu_ref, du_ref, emu_ref, ou_ref),
            (si_ref, d2i_ref, zi_ref, di_ref, emi_ref, oi_ref)):
        zf = z[...] + s[...] * d2[...]
        v = jnp.where(dg[...] > 0.0, zf, em[...])
        n2 = jnp.sum(v * v, axis=1, keepdims=True)
        o[...] = v / jnp.maximum(jnp.sqrt(n2), 1e-12)


TCG = 32
BR = NPAD // TCG
_w = pl.BlockSpec((BR, D), lambda i: (i, 0))
_c = pl.BlockSpec((BR, 1), lambda i: (i, 0))

_prep = pl.pallas_call(
    _prep_body,
    grid=(TCG,),
    in_specs=[_c, _c, _w, _w],
    out_specs=(_w, _w, _c, _c),
    out_shape=(_f32(NPAD, D), _f32(NPAD, D), _f32(NPAD, 1), _f32(NPAD, 1)),
)

_scale = pl.pallas_call(
    _scale_body,
    grid=(TCG,),
    in_specs=[_w, _w, _c, _c, _w, _w],
    out_specs=(_w, _w, _w, _w),
    out_shape=(_f32(NPAD, D), _f32(NPAD, D), _f32(NPAD, D), _f32(NPAD, D)),
)

_final = pl.pallas_call(
    _final_body,
    grid=(TCG,),
    in_specs=[_w, _w, _c, _c, _w, _w, _c, _c, _w, _w],
    out_specs=(_w, _w),
    out_shape=(_f32(NPAD, D), _f32(NPAD, D)),
)


def kernel(edge_index, user_emb, item_emb):
    eu = edge_index[0]
    ei = edge_index[1]
    pad = jnp.full((E_PAD - E,), TRASH, dtype=jnp.int32)
    eu2 = jnp.concatenate([eu, pad]).reshape(NCHUNK, CH)
    ei2 = jnp.concatenate([ei, pad]).reshape(NCHUNK, CH)
    padrows = jnp.zeros((NPAD - NU, D), jnp.float32)
    emu = jnp.concatenate([user_emb, padrows], axis=0)
    emi = jnp.concatenate([item_emb, padrows], axis=0)
    z1 = jnp.zeros((R,), jnp.float32)
    z2 = jnp.zeros((SR, D), jnp.float32)

    du, di = _deg_kernel(eu2, ei2, z1)
    du2 = du.reshape(NPAD, 1)
    di2 = di.reshape(NPAD, 1)
    yu, yi, d2u, d2i = _prep(du2, di2, emu, emi)
    zu, zi = yu, yi
    for _ in range(NLAYERS - 1):
        su, si = _layer_kernel(yu, yi, eu2, ei2, z2)
        yu, yi, zu, zi = _scale(su, si, d2u, d2i, zu, zi)
    su, si = _layer_kernel(yu, yi, eu2, ei2, z2)
    ou, oi = _final(su, si, d2u, d2i, zu, zi, du2, di2, emu, emi)
    return (ou[:NU], oi[:NI])


# speedup vs baseline: 76.9862x; 1.1353x over previous
import functools

import jax
import jax.numpy as jnp
from jax import lax
from jax.experimental import pallas as pl
from jax.experimental.pallas import tpu as pltpu
from jax.experimental.pallas import tpu_sc as plsc

NU = 50000
NI = 50000
D = 32
NLAYERS = 3
E = 1600000

NSUB = 16
NPAD = 50176
TRASH = 50100
R = NPAD // NSUB

CH = 128
GRP = 8
GRPL = 2
CPT = 784
NGRP = CPT // GRP
NGRPL = CPT // GRPL
NCHUNK = NSUB * CPT
E_PAD = NCHUNK * CH
SR = GRPL * CH

_mesh = plsc.VectorSubcoreMesh(core_axis_name="c", subcore_axis_name="s")


def _f32(*shape):
    return jax.ShapeDtypeStruct(shape, jnp.float32)


@functools.partial(
    pl.kernel,
    out_type=(_f32(NPAD), _f32(NPAD)),
    mesh=_mesh,
    scratch_types=[
        pltpu.VMEM((GRP, CH), jnp.int32),
        pltpu.VMEM((CH,), jnp.float32),
        pltpu.VMEM((R,), jnp.float32),
        pltpu.VMEM_SHARED((NPAD,), jnp.float32),
    ],
)
def _deg_kernel(eu_ref, ei_ref, z1_ref, du_ref, di_ref, idx_v, ones_v, zb_v, acc):
    cid = lax.axis_index("c")
    sid = lax.axis_index("s")
    for k in range(CH // 16):
        ones_v[pl.ds(16 * k, 16)] = jnp.ones((16,), jnp.float32)
    pltpu.sync_copy(z1_ref, zb_v)
    pltpu.sync_copy(zb_v, acc.at[pl.ds(sid * R, R)])
    plsc.subcore_barrier()

    def run(e_ref, out_ref):
        def body(g, carry):
            cb = sid * CPT + g * GRP
            pltpu.sync_copy(e_ref.at[pl.ds(cb, GRP)], idx_v)
            for j in range(GRP):
                pltpu.sync_copy(ones_v, acc.at[idx_v.at[j]], add=True)
            return carry

        lax.fori_loop(0, NGRP, body, 0)
        plsc.subcore_barrier()
        pltpu.sync_copy(acc.at[pl.ds(sid * R, R)], zb_v)
        pltpu.sync_copy(zb_v, out_ref.at[pl.ds(sid * R, R)])

    @pl.when(cid == 0)
    def _():
        run(eu_ref, du_ref)

    @pl.when(cid == 1)
    def _():
        run(ei_ref, di_ref)


@functools.partial(
    pl.kernel,
    out_type=(_f32(NPAD, D), _f32(NPAD, D)),
    mesh=_mesh,
    scratch_types=[
        [pltpu.VMEM((GRPL, CH), jnp.int32) for _ in range(3)],
        [pltpu.VMEM((GRPL, CH), jnp.int32) for _ in range(3)],
        [pltpu.VMEM((SR, D), jnp.float32) for _ in range(3)],
        pltpu.VMEM_SHARED((NPAD, D), jnp.float32),
        [pltpu.SemaphoreType.DMA for _ in range(3)],
        [pltpu.SemaphoreType.DMA for _ in range(3)],
    ],
    compiler_params=pltpu.CompilerParams(use_tc_tiling_on_sc=False),
)
def _layer_kernel(yu_ref, yi_ref, eu_ref, ei_ref, z2_ref, su_ref, si_ref,
                  gidx_v, sidx_v, rows_v, acc, gsem, isem):
    cid = lax.axis_index("c")
    sid = lax.axis_index("s")
    nfull, tail = R // SR, R % SR
    pltpu.sync_copy(z2_ref, rows_v[0])
    for k in range(nfull):
        pltpu.sync_copy(rows_v[0], acc.at[pl.ds(sid * R + k * SR, SR)])
    if tail:
        pltpu.sync_copy(rows_v[0].at[pl.ds(0, tail)],
                        acc.at[pl.ds(sid * R + nfull * SR, tail)])
    plsc.subcore_barrier()

    def run(tab_ref, ge_ref, se_ref, out_ref):
        def load_idx(kk, s):
            cb = sid * CPT + kk * GRPL
            pltpu.async_copy(ge_ref.at[pl.ds(cb, GRPL)], gidx_v[s], isem[s])
            pltpu.async_copy(se_ref.at[pl.ds(cb, GRPL)], sidx_v[s], isem[s])

        def wait_idx(s):
            pltpu.make_async_copy(ge_ref.at[pl.ds(0, GRPL)], gidx_v[s], isem[s]).wait()
            pltpu.make_async_copy(ge_ref.at[pl.ds(0, GRPL)], sidx_v[s], isem[s]).wait()

        def fire(s):
            for j in range(GRPL):
                pltpu.async_copy(tab_ref.at[gidx_v[s].at[j]],
                                 rows_v[s].at[pl.ds(j * CH, CH)], gsem[s])

        def drain_and_scatter(s):
            for j in range(GRPL):
                pltpu.make_async_copy(tab_ref.at[gidx_v[s].at[j]],
                                      rows_v[s].at[pl.ds(j * CH, CH)],
                                      gsem[s]).wait()
            for j in range(GRPL):
                pltpu.sync_copy(rows_v[s].at[pl.ds(j * CH, CH)],
                                acc.at[sidx_v[s].at[j]], add=True)

        load_idx(0, 0)
        load_idx(1, 1)
        wait_idx(0)
        fire(0)

        def body(t, carry):
            for d in range(3):
                k = 3 * t + d
                s0, s1, s2 = d, (d + 1) % 3, (d + 2) % 3
                load_idx(k + 2, s2)
                wait_idx(s1)
                fire(s1)
                drain_and_scatter(s0)
            return carry

        lax.fori_loop(0, (NGRPL - 2) // 3, body, 0)
        s0, s1 = (NGRPL - 2) % 3, (NGRPL - 1) % 3
        wait_idx(s1)
        fire(s1)
        drain_and_scatter(s0)
        drain_and_scatter(s1)
        plsc.subcore_barrier()
        for k in range(nfull):
            base = sid * R + k * SR
            pltpu.sync_copy(acc.at[pl.ds(base, SR)], rows_v[k % 3])
            pltpu.sync_copy(rows_v[k % 3], out_ref.at[pl.ds(base, SR)])
        if tail:
            base = sid * R + nfull * SR
            pltpu.sync_copy(acc.at[pl.ds(base, tail)], rows_v[0].at[pl.ds(0, tail)])
            pltpu.sync_copy(rows_v[0].at[pl.ds(0, tail)],
                            out_ref.at[pl.ds(base, tail)])

    @pl.when(cid == 0)
    def _():
        run(yi_ref, ei_ref, eu_ref, su_ref)

    @pl.when(cid == 1)
    def _():
        run(yu_ref, eu_ref, ei_ref, si_ref)


def _prep_body(du_ref, di_ref, emu_ref, emi_ref, yu_ref, yi_ref):
    for dref, eref, yref in ((du_ref, emu_ref, yu_ref),
                             (di_ref, emi_ref, yi_ref)):
        deg = dref[...]
        dinv = jnp.where(deg > 0.0, lax.rsqrt(jnp.maximum(deg, 1.0)), 0.0)
        yref[...] = eref[...] * dinv


def _scale_body(su_ref, si_ref, dbu_ref, dbi_ref, zu_ref, zi_ref,
                yu_o, yi_o, zu_o, zi_o):
    for s, db, z, yo, zo in ((su_ref, dbu_ref, zu_ref, yu_o, zu_o),
                             (si_ref, dbi_ref, zi_ref, yi_o, zi_o)):
        deg = db[...]
        y = s[...] * jnp.where(deg > 0.0, 1.0 / jnp.maximum(deg, 1.0), 0.0)
        yo[...] = y
        zo[...] = z[...] + y


def _final_body(zu_ref, zi_ref, du_ref, di_ref, emu_ref, emi_ref, ou_ref, oi_ref):
    for z, dg, em, o in ((zu_ref, du_ref, emu_ref, ou_ref),
                         (zi_ref, di_ref, emi_ref, oi_ref)):
        v = jnp.where(dg[...] > 0.0, z[...], em[...])
        n2 = jnp.sum(v * v, axis=1, keepdims=True)
        o[...] = v / jnp.maximum(jnp.sqrt(n2), 1e-12)


TCG = 32
BR = NPAD // TCG
N128 = NPAD * D // 128
B128 = N128 // TCG
_w = pl.BlockSpec((BR, D), lambda i: (i, 0))
_c = pl.BlockSpec((BR, 1), lambda i: (i, 0))
_l = pl.BlockSpec((B128, 128), lambda i: (i, 0))

_prep = pl.pallas_call(
    _prep_body,
    grid=(TCG,),
    in_specs=[_c, _c, _w, _w],
    out_specs=(_w, _w),
    out_shape=(_f32(NPAD, D), _f32(NPAD, D)),
)

_scale = pl.pallas_call(
    _scale_body,
    grid=(TCG,),
    in_specs=[_l, _l, _l, _l, _l, _l],
    out_specs=(_l, _l, _l, _l),
    out_shape=(_f32(N128, 128), _f32(N128, 128),
               _f32(N128, 128), _f32(N128, 128)),
)

_final = pl.pallas_call(
    _final_body,
    grid=(TCG,),
    in_specs=[_w, _w, _c, _c, _w, _w],
    out_specs=(_w, _w),
    out_shape=(_f32(NPAD, D), _f32(NPAD, D)),
)


def kernel(edge_index, user_emb, item_emb):
    eu = edge_index[0]
    ei = edge_index[1]
    pad = jnp.full((E_PAD - E,), TRASH, dtype=jnp.int32)
    eu2 = jnp.concatenate([eu, pad]).reshape(NCHUNK, CH)
    ei2 = jnp.concatenate([ei, pad]).reshape(NCHUNK, CH)
    padrows = jnp.zeros((NPAD - NU, D), jnp.float32)
    emu = jnp.concatenate([user_emb, padrows], axis=0)
    emi = jnp.concatenate([item_emb, padrows], axis=0)
    z1 = jnp.zeros((R,), jnp.float32)
    z2 = jnp.zeros((SR, D), jnp.float32)

    du, di = _deg_kernel(eu2, ei2, z1)
    du2 = du.reshape(NPAD, 1)
    di2 = di.reshape(NPAD, 1)
    dbu = jnp.broadcast_to(du2, (NPAD, D)).reshape(N128, 128)
    dbi = jnp.broadcast_to(di2, (NPAD, D)).reshape(N128, 128)
    yu, yi = _prep(du2, di2, emu, emi)
    zu, zi = yu.reshape(N128, 128), yi.reshape(N128, 128)
    for _ in range(NLAYERS):
        su, si = _layer_kernel(yu, yi, eu2, ei2, z2)
        yu128, yi128, zu, zi = _scale(su.reshape(N128, 128),
                                      si.reshape(N128, 128), dbu, dbi, zu, zi)
        yu = yu128.reshape(NPAD, D)
        yi = yi128.reshape(NPAD, D)
    ou, oi = _final(zu.reshape(NPAD, D), zi.reshape(NPAD, D),
                    du2, di2, emu, emi)
    return (ou[:NU], oi[:NI])


# speedup vs baseline: 83.3114x; 1.0822x over previous
import functools

import jax
import jax.numpy as jnp
from jax import lax
from jax.experimental import pallas as pl
from jax.experimental.pallas import tpu as pltpu
from jax.experimental.pallas import tpu_sc as plsc

NU = 50000
NI = 50000
D = 32
NLAYERS = 3
E = 1600000

NSUB = 16
NPAD = 50176
TRASH = 50100
R = NPAD // NSUB

CH = 128
GRP = 8
GRPL = 2
CPT = 784
NGRP = CPT // GRP
NGRPL = CPT // GRPL
NCHUNK = NSUB * CPT
E_PAD = NCHUNK * CH
SR = GRPL * CH

_mesh = plsc.VectorSubcoreMesh(core_axis_name="c", subcore_axis_name="s")


def _f32(*shape):
    return jax.ShapeDtypeStruct(shape, jnp.float32)


@functools.partial(
    pl.kernel,
    out_type=(_f32(NPAD), _f32(NPAD)),
    mesh=_mesh,
    scratch_types=[
        pltpu.VMEM((GRP, CH), jnp.int32),
        pltpu.VMEM((CH,), jnp.float32),
        pltpu.VMEM((R,), jnp.float32),
        pltpu.VMEM_SHARED((NPAD,), jnp.float32),
    ],
)
def _deg_kernel(eu_ref, ei_ref, z1_ref, du_ref, di_ref, idx_v, ones_v, zb_v, acc):
    cid = lax.axis_index("c")
    sid = lax.axis_index("s")
    for k in range(CH // 16):
        ones_v[pl.ds(16 * k, 16)] = jnp.ones((16,), jnp.float32)
    pltpu.sync_copy(z1_ref, zb_v)
    pltpu.sync_copy(zb_v, acc.at[pl.ds(sid * R, R)])
    plsc.subcore_barrier()

    def run(e_ref, out_ref):
        def body(g, carry):
            cb = sid * CPT + g * GRP
            pltpu.sync_copy(e_ref.at[pl.ds(cb, GRP)], idx_v)
            for j in range(GRP):
                pltpu.sync_copy(ones_v, acc.at[idx_v.at[j]], add=True)
            return carry

        lax.fori_loop(0, NGRP, body, 0)
        plsc.subcore_barrier()
        pltpu.sync_copy(acc.at[pl.ds(sid * R, R)], zb_v)
        pltpu.sync_copy(zb_v, out_ref.at[pl.ds(sid * R, R)])

    @pl.when(cid == 0)
    def _():
        run(eu_ref, du_ref)

    @pl.when(cid == 1)
    def _():
        run(ei_ref, di_ref)


@functools.partial(
    pl.kernel,
    out_type=(_f32(NPAD, D), _f32(NPAD, D)),
    mesh=_mesh,
    scratch_types=[
        [pltpu.VMEM((GRPL, CH), jnp.int32) for _ in range(4)],
        [pltpu.VMEM((GRPL, CH), jnp.int32) for _ in range(4)],
        [pltpu.VMEM((SR, D), jnp.float32) for _ in range(3)],
        pltpu.VMEM_SHARED((NPAD, D), jnp.float32),
        [pltpu.SemaphoreType.DMA for _ in range(3)],
        [pltpu.SemaphoreType.DMA for _ in range(4)],
        [pltpu.SemaphoreType.DMA for _ in range(3)],
    ],
    compiler_params=pltpu.CompilerParams(use_tc_tiling_on_sc=False),
)
def _layer_kernel(yu_ref, yi_ref, eu_ref, ei_ref, z2_ref, su_ref, si_ref,
                  gidx_v, sidx_v, rows_v, acc, gsem, isem, ssem):
    cid = lax.axis_index("c")
    sid = lax.axis_index("s")
    nfull, tail = R // SR, R % SR
    pltpu.sync_copy(z2_ref, rows_v[0])
    for k in range(nfull):
        pltpu.sync_copy(rows_v[0], acc.at[pl.ds(sid * R + k * SR, SR)])
    if tail:
        pltpu.sync_copy(rows_v[0].at[pl.ds(0, tail)],
                        acc.at[pl.ds(sid * R + nfull * SR, tail)])
    plsc.subcore_barrier()

    def run(tab_ref, ge_ref, se_ref, out_ref):
        def load_idx(kk, si):
            cb = sid * CPT + kk * GRPL
            pltpu.async_copy(ge_ref.at[pl.ds(cb, GRPL)], gidx_v[si], isem[si])
            pltpu.async_copy(se_ref.at[pl.ds(cb, GRPL)], sidx_v[si], isem[si])

        def wait_idx(si):
            pltpu.make_async_copy(ge_ref.at[pl.ds(0, GRPL)], gidx_v[si], isem[si]).wait()
            pltpu.make_async_copy(ge_ref.at[pl.ds(0, GRPL)], sidx_v[si], isem[si]).wait()

        def fire(sr, si):
            for j in range(GRPL):
                pltpu.async_copy(tab_ref.at[gidx_v[si].at[j]],
                                 rows_v[sr].at[pl.ds(j * CH, CH)], gsem[sr])

        def wait_gather(sr, si):
            for j in range(GRPL):
                pltpu.make_async_copy(tab_ref.at[gidx_v[si].at[j]],
                                      rows_v[sr].at[pl.ds(j * CH, CH)],
                                      gsem[sr]).wait()

        def fire_scatter(sr, si):
            for j in range(GRPL):
                pltpu.async_copy(rows_v[sr].at[pl.ds(j * CH, CH)],
                                 acc.at[sidx_v[si].at[j]], ssem[sr], add=True)

        def wait_scatter(sr, si):
            for j in range(GRPL):
                pltpu.make_async_copy(rows_v[sr].at[pl.ds(j * CH, CH)],
                                      acc.at[sidx_v[si].at[j]],
                                      ssem[sr]).wait()

        def step(k, km, first=False, fire_next_idx=True, fire_next_gather=True):
            if not first:
                wait_scatter((km - 2) % 3, (km - 2) % 4)
            if fire_next_idx:
                load_idx(k + 2, (km + 2) % 4)
            if fire_next_gather:
                wait_idx((km + 1) % 4)
                fire((km + 1) % 3, (km + 1) % 4)
            wait_gather(km % 3, km % 4)
            fire_scatter(km % 3, km % 4)

        load_idx(0, 0)
        load_idx(1, 1)
        wait_idx(0)
        fire(0, 0)
        step(0, 0, first=True)
        step(1, 1, first=True)

        def body(t, carry):
            for d in range(12):
                step(2 + 12 * t + d, 2 + d)
            return carry

        lax.fori_loop(0, (NGRPL - 8) // 12, body, 0)
        for k in range(NGRPL - 6, NGRPL):
            step(k, k, fire_next_idx=(k + 2 < NGRPL),
                 fire_next_gather=(k + 1 < NGRPL))
        wait_scatter((NGRPL - 2) % 3, (NGRPL - 2) % 4)
        wait_scatter((NGRPL - 1) % 3, (NGRPL - 1) % 4)
        plsc.subcore_barrier()
        for k in range(nfull):
            base = sid * R + k * SR
            pltpu.sync_copy(acc.at[pl.ds(base, SR)], rows_v[k % 3])
            pltpu.sync_copy(rows_v[k % 3], out_ref.at[pl.ds(base, SR)])
        if tail:
            base = sid * R + nfull * SR
            pltpu.sync_copy(acc.at[pl.ds(base, tail)], rows_v[0].at[pl.ds(0, tail)])
            pltpu.sync_copy(rows_v[0].at[pl.ds(0, tail)],
                            out_ref.at[pl.ds(base, tail)])

    @pl.when(cid == 0)
    def _():
        run(yi_ref, ei_ref, eu_ref, su_ref)

    @pl.when(cid == 1)
    def _():
        run(yu_ref, eu_ref, ei_ref, si_ref)


def _prep_body(du_ref, di_ref, emu_ref, emi_ref, yu_ref, yi_ref):
    for dref, eref, yref in ((du_ref, emu_ref, yu_ref),
                             (di_ref, emi_ref, yi_ref)):
        deg = dref[...]
        dinv = jnp.where(deg > 0.0, lax.rsqrt(jnp.maximum(deg, 1.0)), 0.0)
        yref[...] = eref[...] * dinv


def _scale_body(su_ref, si_ref, dbu_ref, dbi_ref, zu_ref, zi_ref,
                yu_o, yi_o, zu_o, zi_o):
    for s, db, z, yo, zo in ((su_ref, dbu_ref, zu_ref, yu_o, zu_o),
                             (si_ref, dbi_ref, zi_ref, yi_o, zi_o)):
        deg = db[...]
        y = s[...] * jnp.where(deg > 0.0, 1.0 / jnp.maximum(deg, 1.0), 0.0)
        yo[...] = y
        zo[...] = z[...] + y


def _final_body(zu_ref, zi_ref, du_ref, di_ref, emu_ref, emi_ref, ou_ref, oi_ref):
    for z, dg, em, o in ((zu_ref, du_ref, emu_ref, ou_ref),
                         (zi_ref, di_ref, emi_ref, oi_ref)):
        v = jnp.where(dg[...] > 0.0, z[...], em[...])
        n2 = jnp.sum(v * v, axis=1, keepdims=True)
        o[...] = v / jnp.maximum(jnp.sqrt(n2), 1e-12)


TCG = 32
BR = NPAD // TCG
N128 = NPAD * D // 128
B128 = N128 // TCG
_w = pl.BlockSpec((BR, D), lambda i: (i, 0))
_c = pl.BlockSpec((BR, 1), lambda i: (i, 0))
_l = pl.BlockSpec((B128, 128), lambda i: (i, 0))

_prep = pl.pallas_call(
    _prep_body,
    grid=(TCG,),
    in_specs=[_c, _c, _w, _w],
    out_specs=(_w, _w),
    out_shape=(_f32(NPAD, D), _f32(NPAD, D)),
)

_scale = pl.pallas_call(
    _scale_body,
    grid=(TCG,),
    in_specs=[_l, _l, _l, _l, _l, _l],
    out_specs=(_l, _l, _l, _l),
    out_shape=(_f32(N128, 128), _f32(N128, 128),
               _f32(N128, 128), _f32(N128, 128)),
)

_final = pl.pallas_call(
    _final_body,
    grid=(TCG,),
    in_specs=[_w, _w, _c, _c, _w, _w],
    out_specs=(_w, _w),
    out_shape=(_f32(NPAD, D), _f32(NPAD, D)),
)


def kernel(edge_index, user_emb, item_emb):
    eu = edge_index[0]
    ei = edge_index[1]
    pad = jnp.full((E_PAD - E,), TRASH, dtype=jnp.int32)
    eu2 = jnp.concatenate([eu, pad]).reshape(NCHUNK, CH)
    ei2 = jnp.concatenate([ei, pad]).reshape(NCHUNK, CH)
    padrows = jnp.zeros((NPAD - NU, D), jnp.float32)
    emu = jnp.concatenate([user_emb, padrows], axis=0)
    emi = jnp.concatenate([item_emb, padrows], axis=0)
    z1 = jnp.zeros((R,), jnp.float32)
    z2 = jnp.zeros((SR, D), jnp.float32)

    du, di = _deg_kernel(eu2, ei2, z1)
    du2 = du.reshape(NPAD, 1)
    di2 = di.reshape(NPAD, 1)
    dbu = jnp.broadcast_to(du2, (NPAD, D)).reshape(N128, 128)
    dbi = jnp.broadcast_to(di2, (NPAD, D)).reshape(N128, 128)
    yu, yi = _prep(du2, di2, emu, emi)
    zu, zi = yu.reshape(N128, 128), yi.reshape(N128, 128)
    for _ in range(NLAYERS):
        su, si = _layer_kernel(yu, yi, eu2, ei2, z2)
        yu128, yi128, zu, zi = _scale(su.reshape(N128, 128),
                                      si.reshape(N128, 128), dbu, dbi, zu, zi)
        yu = yu128.reshape(NPAD, D)
        yi = yi128.reshape(NPAD, D)
    ou, oi = _final(zu.reshape(NPAD, D), zi.reshape(NPAD, D),
                    du2, di2, emu, emi)
    return (ou[:NU], oi[:NI])


# speedup vs baseline: 88.4196x; 1.0613x over previous
import functools

import jax
import jax.numpy as jnp
from jax import lax
from jax.experimental import pallas as pl
from jax.experimental.pallas import tpu as pltpu
from jax.experimental.pallas import tpu_sc as plsc

NU = 50000
NI = 50000
D = 32
NLAYERS = 3
E = 1600000

NSUB = 16
NPAD = 50176
TRASH = 50100
R = NPAD // NSUB

CH = 128
GRP = 8
GRPL = 2
CPT = 784
NGRP = CPT // GRP
NGRPL = CPT // GRPL
NCHUNK = NSUB * CPT
E_PAD = NCHUNK * CH
SR = GRPL * CH

_mesh = plsc.VectorSubcoreMesh(core_axis_name="c", subcore_axis_name="s")


def _f32(*shape):
    return jax.ShapeDtypeStruct(shape, jnp.float32)


@functools.partial(
    pl.kernel,
    out_type=(_f32(NPAD), _f32(NPAD)),
    mesh=_mesh,
    scratch_types=[
        pltpu.VMEM((GRP, CH), jnp.int32),
        pltpu.VMEM((CH,), jnp.float32),
        pltpu.VMEM((R,), jnp.float32),
        pltpu.VMEM_SHARED((NPAD,), jnp.float32),
    ],
)
def _deg_kernel(eu_ref, ei_ref, z1_ref, du_ref, di_ref, idx_v, ones_v, zb_v, acc):
    cid = lax.axis_index("c")
    sid = lax.axis_index("s")
    for k in range(CH // 16):
        ones_v[pl.ds(16 * k, 16)] = jnp.ones((16,), jnp.float32)
    pltpu.sync_copy(z1_ref, zb_v)
    pltpu.sync_copy(zb_v, acc.at[pl.ds(sid * R, R)])
    plsc.subcore_barrier()

    def run(e_ref, out_ref):
        def body(g, carry):
            cb = sid * CPT + g * GRP
            pltpu.sync_copy(e_ref.at[pl.ds(cb, GRP)], idx_v)
            for j in range(GRP):
                pltpu.sync_copy(ones_v, acc.at[idx_v.at[j]], add=True)
            return carry

        lax.fori_loop(0, NGRP, body, 0)
        plsc.subcore_barrier()
        pltpu.sync_copy(acc.at[pl.ds(sid * R, R)], zb_v)
        pltpu.sync_copy(zb_v, out_ref.at[pl.ds(sid * R, R)])

    @pl.when(cid == 0)
    def _():
        run(eu_ref, du_ref)

    @pl.when(cid == 1)
    def _():
        run(ei_ref, di_ref)


@functools.partial(
    pl.kernel,
    out_type=(_f32(NPAD, D), _f32(NPAD, D)),
    mesh=_mesh,
    scratch_types=[
        [pltpu.VMEM((GRPL, CH), jnp.int32) for _ in range(4)],
        [pltpu.VMEM((GRPL, CH), jnp.int32) for _ in range(4)],
        [pltpu.VMEM((SR, D), jnp.float32) for _ in range(3)],
        pltpu.VMEM_SHARED((NPAD, D), jnp.float32),
        [pltpu.SemaphoreType.DMA for _ in range(3)],
        [pltpu.SemaphoreType.DMA for _ in range(4)],
        [pltpu.SemaphoreType.DMA for _ in range(3)],
    ],
    compiler_params=pltpu.CompilerParams(use_tc_tiling_on_sc=False),
)
def _layer_kernel(yu_ref, yi_ref, eu_ref, ei_ref, z2_ref, su_ref, si_ref,
                  gidx_v, sidx_v, rows_v, acc, gsem, isem, ssem):
    cid = lax.axis_index("c")
    sid = lax.axis_index("s")
    nfull, tail = R // SR, R % SR
    pltpu.sync_copy(z2_ref, rows_v[0])
    for k in range(nfull):
        pltpu.sync_copy(rows_v[0], acc.at[pl.ds(sid * R + k * SR, SR)])
    if tail:
        pltpu.sync_copy(rows_v[0].at[pl.ds(0, tail)],
                        acc.at[pl.ds(sid * R + nfull * SR, tail)])
    plsc.subcore_barrier()

    def run(tab_ref, ge_ref, se_ref, out_ref):
        def load_idx(kk, si):
            cb = sid * CPT + kk * GRPL
            pltpu.async_copy(ge_ref.at[pl.ds(cb, GRPL)], gidx_v[si], isem[si])
            pltpu.async_copy(se_ref.at[pl.ds(cb, GRPL)], sidx_v[si], isem[si])

        def wait_idx(si):
            pltpu.make_async_copy(ge_ref.at[pl.ds(0, GRPL)], gidx_v[si], isem[si]).wait()
            pltpu.make_async_copy(ge_ref.at[pl.ds(0, GRPL)], sidx_v[si], isem[si]).wait()

        def fire(sr, si):
            for j in range(GRPL):
                pltpu.async_copy(tab_ref.at[gidx_v[si].at[j]],
                                 rows_v[sr].at[pl.ds(j * CH, CH)], gsem[sr])

        def wait_gather(sr, si):
            for j in range(GRPL):
                pltpu.make_async_copy(tab_ref.at[gidx_v[si].at[j]],
                                      rows_v[sr].at[pl.ds(j * CH, CH)],
                                      gsem[sr]).wait()

        def fire_scatter(sr, si):
            for j in range(GRPL):
                pltpu.async_copy(rows_v[sr].at[pl.ds(j * CH, CH)],
                                 acc.at[sidx_v[si].at[j]], ssem[sr], add=True)

        def wait_scatter(sr, si):
            for j in range(GRPL):
                pltpu.make_async_copy(rows_v[sr].at[pl.ds(j * CH, CH)],
                                      acc.at[sidx_v[si].at[j]],
                                      ssem[sr]).wait()

        def step(k, km, first=False, fire_next_idx=True, fire_next_gather=True):
            if not first:
                wait_scatter((km - 2) % 3, (km - 2) % 4)
            if fire_next_idx:
                load_idx(k + 2, (km + 2) % 4)
            if fire_next_gather:
                wait_idx((km + 1) % 4)
                fire((km + 1) % 3, (km + 1) % 4)
            wait_gather(km % 3, km % 4)
            fire_scatter(km % 3, km % 4)

        load_idx(0, 0)
        load_idx(1, 1)
        wait_idx(0)
        fire(0, 0)
        step(0, 0, first=True)
        step(1, 1, first=True)

        def body(t, carry):
            for d in range(12):
                step(2 + 12 * t + d, 2 + d)
            return carry

        lax.fori_loop(0, (NGRPL - 8) // 12, body, 0)
        for k in range(NGRPL - 6, NGRPL):
            step(k, k, fire_next_idx=(k + 2 < NGRPL),
                 fire_next_gather=(k + 1 < NGRPL))
        wait_scatter((NGRPL - 2) % 3, (NGRPL - 2) % 4)
        wait_scatter((NGRPL - 1) % 3, (NGRPL - 1) % 4)
        plsc.subcore_barrier()
        for k in range(nfull):
            base = sid * R + k * SR
            pltpu.sync_copy(acc.at[pl.ds(base, SR)], rows_v[k % 3])
            pltpu.sync_copy(rows_v[k % 3], out_ref.at[pl.ds(base, SR)])
        if tail:
            base = sid * R + nfull * SR
            pltpu.sync_copy(acc.at[pl.ds(base, tail)], rows_v[0].at[pl.ds(0, tail)])
            pltpu.sync_copy(rows_v[0].at[pl.ds(0, tail)],
                            out_ref.at[pl.ds(base, tail)])

    @pl.when(cid == 0)
    def _():
        run(yi_ref, ei_ref, eu_ref, su_ref)

    @pl.when(cid == 1)
    def _():
        run(yu_ref, eu_ref, ei_ref, si_ref)


def _prep_body(dbu_ref, dbi_ref, emu_ref, emi_ref, yu_ref, yi_ref):
    for dref, eref, yref in ((dbu_ref, emu_ref, yu_ref),
                             (dbi_ref, emi_ref, yi_ref)):
        deg = dref[...]
        dinv = jnp.where(deg > 0.0, lax.rsqrt(jnp.maximum(deg, 1.0)), 0.0)
        yref[...] = eref[...] * dinv


def _scale_body(su_ref, si_ref, dbu_ref, dbi_ref, zu_ref, zi_ref,
                yu_o, yi_o, zu_o, zi_o):
    for s, db, z, yo, zo in ((su_ref, dbu_ref, zu_ref, yu_o, zu_o),
                             (si_ref, dbi_ref, zi_ref, yi_o, zi_o)):
        deg = db[...]
        y = s[...] * jnp.where(deg > 0.0, 1.0 / jnp.maximum(deg, 1.0), 0.0)
        yo[...] = y
        zo[...] = z[...] + y


def _final_body(zu_ref, zi_ref, dbu_ref, dbi_ref, emu_ref, emi_ref, ou_ref, oi_ref):
    ii = lax.broadcasted_iota(jnp.int32, (128, 128), 0) // D
    jj = lax.broadcasted_iota(jnp.int32, (128, 128), 1) // D
    mask = (ii == jj).astype(jnp.float32)
    for z, dg, em, o in ((zu_ref, dbu_ref, emu_ref, ou_ref),
                         (zi_ref, dbi_ref, emi_ref, oi_ref)):
        v = jnp.where(dg[...] > 0.0, z[...], em[...])
        n2 = jnp.dot(v * v, mask, preferred_element_type=jnp.float32)
        o[...] = v / jnp.maximum(jnp.sqrt(n2), 1e-12)


TCG = 32
BR = NPAD // TCG
N128 = NPAD * D // 128
B128 = N128 // TCG
_w = pl.BlockSpec((BR, D), lambda i: (i, 0))
_c = pl.BlockSpec((BR, 1), lambda i: (i, 0))
_l = pl.BlockSpec((B128, 128), lambda i: (i, 0))

_prep = pl.pallas_call(
    _prep_body,
    grid=(TCG,),
    in_specs=[_l, _l, _l, _l],
    out_specs=(_l, _l),
    out_shape=(_f32(N128, 128), _f32(N128, 128)),
)

_scale = pl.pallas_call(
    _scale_body,
    grid=(TCG,),
    in_specs=[_l, _l, _l, _l, _l, _l],
    out_specs=(_l, _l, _l, _l),
    out_shape=(_f32(N128, 128), _f32(N128, 128),
               _f32(N128, 128), _f32(N128, 128)),
)

_final = pl.pallas_call(
    _final_body,
    grid=(TCG,),
    in_specs=[_l, _l, _l, _l, _l, _l],
    out_specs=(_l, _l),
    out_shape=(_f32(N128, 128), _f32(N128, 128)),
)


def kernel(edge_index, user_emb, item_emb):
    eu = edge_index[0]
    ei = edge_index[1]
    pad = jnp.full((E_PAD - E,), TRASH, dtype=jnp.int32)
    eu2 = jnp.concatenate([eu, pad]).reshape(NCHUNK, CH)
    ei2 = jnp.concatenate([ei, pad]).reshape(NCHUNK, CH)
    padrows = jnp.zeros((N128 - NU * D // 128, 128), jnp.float32)
    emu = jnp.concatenate([user_emb.reshape(NU * D // 128, 128), padrows], axis=0)
    emi = jnp.concatenate([item_emb.reshape(NI * D // 128, 128), padrows], axis=0)
    z1 = jnp.zeros((R,), jnp.float32)
    z2 = jnp.zeros((SR, D), jnp.float32)

    du, di = _deg_kernel(eu2, ei2, z1)
    dbu = jnp.broadcast_to(du.reshape(NPAD, 1), (NPAD, D)).reshape(N128, 128)
    dbi = jnp.broadcast_to(di.reshape(NPAD, 1), (NPAD, D)).reshape(N128, 128)
    zu, zi = _prep(dbu, dbi, emu, emi)
    yu, yi = zu.reshape(NPAD, D), zi.reshape(NPAD, D)
    for _ in range(NLAYERS):
        su, si = _layer_kernel(yu, yi, eu2, ei2, z2)
        yu128, yi128, zu, zi = _scale(su.reshape(N128, 128),
                                      si.reshape(N128, 128), dbu, dbi, zu, zi)
        yu = yu128.reshape(NPAD, D)
        yi = yi128.reshape(NPAD, D)
    ou, oi = _final(zu, zi, dbu, dbi, emu, emi)
    return (ou.reshape(NPAD, D)[:NU], oi.reshape(NPAD, D)[:NI])


# speedup vs baseline: 91.9520x; 1.0400x over previous
import functools

import jax
import jax.numpy as jnp
from jax import lax
from jax.experimental import pallas as pl
from jax.experimental.pallas import tpu as pltpu
from jax.experimental.pallas import tpu_sc as plsc

NU = 50000
NI = 50000
D = 32
NLAYERS = 3
E = 1600000

NSUB = 16
NPAD = 50176
TRASH = 50100
R = NPAD // NSUB

CH = 128
GRP = 8
GRPL = 2
CPT = 784
NGRP = CPT // GRP
NGRPL = CPT // GRPL
NCHUNK = NSUB * CPT
E_PAD = NCHUNK * CH
SR = GRPL * CH

_mesh = plsc.VectorSubcoreMesh(core_axis_name="c", subcore_axis_name="s")


def _f32(*shape):
    return jax.ShapeDtypeStruct(shape, jnp.float32)


@functools.partial(
    pl.kernel,
    out_type=(_f32(NPAD), _f32(NPAD)),
    mesh=_mesh,
    scratch_types=[
        [pltpu.VMEM((GRP, CH), jnp.int32) for _ in range(2)],
        pltpu.VMEM((CH,), jnp.float32),
        pltpu.VMEM((R,), jnp.float32),
        pltpu.VMEM_SHARED((NPAD,), jnp.float32),
        [pltpu.SemaphoreType.DMA for _ in range(2)],
    ],
)
def _deg_kernel(eu_ref, ei_ref, z1_ref, du_ref, di_ref, idx_v, ones_v, zb_v,
                acc, isem):
    cid = lax.axis_index("c")
    sid = lax.axis_index("s")
    for k in range(CH // 16):
        ones_v[pl.ds(16 * k, 16)] = jnp.ones((16,), jnp.float32)
    pltpu.sync_copy(z1_ref, zb_v)
    pltpu.sync_copy(zb_v, acc.at[pl.ds(sid * R, R)])
    plsc.subcore_barrier()

    def run(e_ref, out_ref):
        def load_idx(g, s):
            cb = sid * CPT + g * GRP
            pltpu.async_copy(e_ref.at[pl.ds(cb, GRP)], idx_v[s], isem[s])

        def half(g, s):
            @pl.when(g < NGRP - 1)
            def _():
                load_idx(g + 1, (s + 1) % 2)

            pltpu.make_async_copy(e_ref.at[pl.ds(0, GRP)], idx_v[s],
                                  isem[s]).wait()
            for j in range(GRP):
                pltpu.sync_copy(ones_v, acc.at[idx_v[s].at[j]], add=True)

        load_idx(0, 0)

        def body(t, carry):
            half(2 * t, 0)
            half(2 * t + 1, 1)
            return carry

        lax.fori_loop(0, NGRP // 2, body, 0)
        plsc.subcore_barrier()
        pltpu.sync_copy(acc.at[pl.ds(sid * R, R)], zb_v)
        pltpu.sync_copy(zb_v, out_ref.at[pl.ds(sid * R, R)])

    @pl.when(cid == 0)
    def _():
        run(eu_ref, du_ref)

    @pl.when(cid == 1)
    def _():
        run(ei_ref, di_ref)


@functools.partial(
    pl.kernel,
    out_type=(_f32(NPAD, D), _f32(NPAD, D)),
    mesh=_mesh,
    scratch_types=[
        [pltpu.VMEM((GRPL, CH), jnp.int32) for _ in range(4)],
        [pltpu.VMEM((GRPL, CH), jnp.int32) for _ in range(4)],
        [pltpu.VMEM((SR, D), jnp.float32) for _ in range(3)],
        pltpu.VMEM_SHARED((NPAD, D), jnp.float32),
        [pltpu.SemaphoreType.DMA for _ in range(3)],
        [pltpu.SemaphoreType.DMA for _ in range(4)],
        [pltpu.SemaphoreType.DMA for _ in range(3)],
    ],
    compiler_params=pltpu.CompilerParams(use_tc_tiling_on_sc=False),
)
def _layer_kernel(yu_ref, yi_ref, eu_ref, ei_ref, z2_ref, su_ref, si_ref,
                  gidx_v, sidx_v, rows_v, acc, gsem, isem, ssem):
    cid = lax.axis_index("c")
    sid = lax.axis_index("s")
    nfull, tail = R // SR, R % SR
    pltpu.sync_copy(z2_ref, rows_v[0])
    for k in range(nfull):
        pltpu.sync_copy(rows_v[0], acc.at[pl.ds(sid * R + k * SR, SR)])
    if tail:
        pltpu.sync_copy(rows_v[0].at[pl.ds(0, tail)],
                        acc.at[pl.ds(sid * R + nfull * SR, tail)])
    plsc.subcore_barrier()

    def run(tab_ref, ge_ref, se_ref, out_ref):
        def load_idx(kk, si):
            cb = sid * CPT + kk * GRPL
            pltpu.async_copy(ge_ref.at[pl.ds(cb, GRPL)], gidx_v[si], isem[si])
            pltpu.async_copy(se_ref.at[pl.ds(cb, GRPL)], sidx_v[si], isem[si])

        def wait_idx(si):
            pltpu.make_async_copy(ge_ref.at[pl.ds(0, GRPL)], gidx_v[si], isem[si]).wait()
            pltpu.make_async_copy(ge_ref.at[pl.ds(0, GRPL)], sidx_v[si], isem[si]).wait()

        def fire(sr, si):
            for j in range(GRPL):
                pltpu.async_copy(tab_ref.at[gidx_v[si].at[j]],
                                 rows_v[sr].at[pl.ds(j * CH, CH)], gsem[sr])

        def wait_gather(sr, si):
            for j in range(GRPL):
                pltpu.make_async_copy(tab_ref.at[gidx_v[si].at[j]],
                                      rows_v[sr].at[pl.ds(j * CH, CH)],
                                      gsem[sr]).wait()

        def fire_scatter(sr, si):
            for j in range(GRPL):
                pltpu.async_copy(rows_v[sr].at[pl.ds(j * CH, CH)],
                                 acc.at[sidx_v[si].at[j]], ssem[sr], add=True)

        def wait_scatter(sr, si):
            for j in range(GRPL):
                pltpu.make_async_copy(rows_v[sr].at[pl.ds(j * CH, CH)],
                                      acc.at[sidx_v[si].at[j]],
                                      ssem[sr]).wait()

        def step(k, km, first=False, fire_next_idx=True, fire_next_gather=True):
            if not first:
                wait_scatter((km - 2) % 3, (km - 2) % 4)
            if fire_next_idx:
                load_idx(k + 2, (km + 2) % 4)
            if fire_next_gather:
                wait_idx((km + 1) % 4)
                fire((km + 1) % 3, (km + 1) % 4)
            wait_gather(km % 3, km % 4)
            fire_scatter(km % 3, km % 4)

        load_idx(0, 0)
        load_idx(1, 1)
        wait_idx(0)
        fire(0, 0)
        step(0, 0, first=True)
        step(1, 1, first=True)

        def body(t, carry):
            for d in range(12):
                step(2 + 12 * t + d, 2 + d)
            return carry

        lax.fori_loop(0, (NGRPL - 8) // 12, body, 0)
        for k in range(NGRPL - 6, NGRPL):
            step(k, k, fire_next_idx=(k + 2 < NGRPL),
                 fire_next_gather=(k + 1 < NGRPL))
        wait_scatter((NGRPL - 2) % 3, (NGRPL - 2) % 4)
        wait_scatter((NGRPL - 1) % 3, (NGRPL - 1) % 4)
        plsc.subcore_barrier()
        for k in range(nfull):
            base = sid * R + k * SR
            pltpu.sync_copy(acc.at[pl.ds(base, SR)], rows_v[k % 3])
            pltpu.sync_copy(rows_v[k % 3], out_ref.at[pl.ds(base, SR)])
        if tail:
            base = sid * R + nfull * SR
            pltpu.sync_copy(acc.at[pl.ds(base, tail)], rows_v[0].at[pl.ds(0, tail)])
            pltpu.sync_copy(rows_v[0].at[pl.ds(0, tail)],
                            out_ref.at[pl.ds(base, tail)])

    @pl.when(cid == 0)
    def _():
        run(yi_ref, ei_ref, eu_ref, su_ref)

    @pl.when(cid == 1)
    def _():
        run(yu_ref, eu_ref, ei_ref, si_ref)


def _prep_body(dbu_ref, dbi_ref, emu_ref, emi_ref, yu_ref, yi_ref):
    for dref, eref, yref in ((dbu_ref, emu_ref, yu_ref),
                             (dbi_ref, emi_ref, yi_ref)):
        deg = dref[...]
        dinv = jnp.where(deg > 0.0, lax.rsqrt(jnp.maximum(deg, 1.0)), 0.0)
        yref[...] = eref[...] * dinv


def _scale_body(su_ref, si_ref, dbu_ref, dbi_ref, zu_ref, zi_ref,
                yu_o, yi_o, zu_o, zi_o):
    for s, db, z, yo, zo in ((su_ref, dbu_ref, zu_ref, yu_o, zu_o),
                             (si_ref, dbi_ref, zi_ref, yi_o, zi_o)):
        deg = db[...]
        y = s[...] * jnp.where(deg > 0.0, 1.0 / jnp.maximum(deg, 1.0), 0.0)
        yo[...] = y
        zo[...] = z[...] + y


def _final_body(zu_ref, zi_ref, dbu_ref, dbi_ref, emu_ref, emi_ref, ou_ref, oi_ref):
    ii = lax.broadcasted_iota(jnp.int32, (128, 128), 0) // D
    jj = lax.broadcasted_iota(jnp.int32, (128, 128), 1) // D
    mask = (ii == jj).astype(jnp.float32)
    for z, dg, em, o in ((zu_ref, dbu_ref, emu_ref, ou_ref),
                         (zi_ref, dbi_ref, emi_ref, oi_ref)):
        v = jnp.where(dg[...] > 0.0, z[...], em[...])
        n2 = jnp.dot(v * v, mask, preferred_element_type=jnp.float32)
        o[...] = v / jnp.maximum(jnp.sqrt(n2), 1e-12)


TCG = 32
BR = NPAD // TCG
N128 = NPAD * D // 128
B128 = N128 // TCG
_w = pl.BlockSpec((BR, D), lambda i: (i, 0))
_c = pl.BlockSpec((BR, 1), lambda i: (i, 0))
_l = pl.BlockSpec((B128, 128), lambda i: (i, 0))

_prep = pl.pallas_call(
    _prep_body,
    grid=(TCG,),
    in_specs=[_l, _l, _l, _l],
    out_specs=(_l, _l),
    out_shape=(_f32(N128, 128), _f32(N128, 128)),
)

_scale = pl.pallas_call(
    _scale_body,
    grid=(TCG,),
    in_specs=[_l, _l, _l, _l, _l, _l],
    out_specs=(_l, _l, _l, _l),
    out_shape=(_f32(N128, 128), _f32(N128, 128),
               _f32(N128, 128), _f32(N128, 128)),
)

_final = pl.pallas_call(
    _final_body,
    grid=(TCG,),
    in_specs=[_l, _l, _l, _l, _l, _l],
    out_specs=(_l, _l),
    out_shape=(_f32(N128, 128), _f32(N128, 128)),
)


def kernel(edge_index, user_emb, item_emb):
    eu = edge_index[0]
    ei = edge_index[1]
    pad = jnp.full((E_PAD - E,), TRASH, dtype=jnp.int32)
    eu2 = jnp.concatenate([eu, pad]).reshape(NCHUNK, CH)
    ei2 = jnp.concatenate([ei, pad]).reshape(NCHUNK, CH)
    padrows = jnp.zeros((N128 - NU * D // 128, 128), jnp.float32)
    emu = jnp.concatenate([user_emb.reshape(NU * D // 128, 128), padrows], axis=0)
    emi = jnp.concatenate([item_emb.reshape(NI * D // 128, 128), padrows], axis=0)
    z1 = jnp.zeros((R,), jnp.float32)
    z2 = jnp.zeros((SR, D), jnp.float32)

    du, di = _deg_kernel(eu2, ei2, z1)
    dbu = jnp.broadcast_to(du.reshape(NPAD, 1), (NPAD, D)).reshape(N128, 128)
    dbi = jnp.broadcast_to(di.reshape(NPAD, 1), (NPAD, D)).reshape(N128, 128)
    zu, zi = _prep(dbu, dbi, emu, emi)
    yu, yi = zu.reshape(NPAD, D), zi.reshape(NPAD, D)
    for _ in range(NLAYERS):
        su, si = _layer_kernel(yu, yi, eu2, ei2, z2)
        yu128, yi128, zu, zi = _scale(su.reshape(N128, 128),
                                      si.reshape(N128, 128), dbu, dbi, zu, zi)
        yu = yu128.reshape(NPAD, D)
        yi = yi128.reshape(NPAD, D)
    ou, oi = _final(zu, zi, dbu, dbi, emu, emi)
    return (ou.reshape(NPAD, D)[:NU], oi.reshape(NPAD, D)[:NI])
